# Initial kernel scaffold; baseline (speedup 1.0000x reference)
import functools

import jax
import jax.numpy as jnp
from jax import lax
from jax.experimental import pallas as pl
from jax.experimental.pallas import tpu as pltpu
from jax.experimental.pallas import tpu_sc as plsc

NC = 2
NS = 16
L = 16
CB = 128


def _sc_mesh():
    return plsc.VectorSubcoreMesh(core_axis_name="c", subcore_axis_name="s")


def _zero_rows(ref, n_rows, n_cols):
    zeros = jnp.zeros((L,), jnp.float32)

    def body(i, carry):
        for j in range(n_cols // L):
            ref[i, pl.ds(j * L, L)] = zeros
        return carry

    lax.fori_loop(0, n_rows, body, 0)


def _make_deg_kernel(n_nodes, n_edges_pad):
    e_worker = n_edges_pad // (NC * NS)
    n_chunks = e_worker // CB
    r_tile = n_nodes // NS

    @functools.partial(
        pl.kernel,
        out_type=jax.ShapeDtypeStruct((NC, n_nodes, L), jnp.float32),
        mesh=_sc_mesh(),
        scratch_types=[
            pltpu.VMEM((CB,), jnp.int32),
            pltpu.VMEM((CB,), jnp.float32),
            pltpu.VMEM((CB, L), jnp.float32),
            pltpu.VMEM((r_tile, L), jnp.float32),
            pltpu.VMEM_SHARED((n_nodes, L), jnp.float32),
        ],
    )
    def deg_kernel(dst_hbm, ew_hbm, out_hbm, dst_v, ew_v, row_v, z_v, acc_sh):
        c = lax.axis_index("c")
        s = lax.axis_index("s")
        _zero_rows(z_v, r_tile, L)
        _zero_rows(row_v, CB, L)
        pltpu.sync_copy(z_v, acc_sh.at[pl.ds(s * r_tile, r_tile)])
        plsc.subcore_barrier()

        base = (c * NS + s) * e_worker
        lane = jnp.arange(L, dtype=jnp.int32)
        zlane = jnp.zeros((L,), jnp.int32)

        def chunk(k, carry):
            off = base + k * CB
            pltpu.sync_copy(dst_hbm.at[pl.ds(off, CB)], dst_v)
            pltpu.sync_copy(ew_hbm.at[pl.ds(off, CB)], ew_v)
            for g in range(CB // L):
                ewg = ew_v[pl.ds(g * L, L)]
                plsc.store_scatter(row_v, [lane + g * L, zlane], ewg)
            pltpu.sync_copy(row_v, acc_sh.at[dst_v], add=True)
            return carry

        lax.fori_loop(0, n_chunks, chunk, 0)
        plsc.subcore_barrier()
        pltpu.sync_copy(
            acc_sh.at[pl.ds(s * r_tile, r_tile)],
            out_hbm.at[c].at[pl.ds(s * r_tile, r_tile)],
        )

    return deg_kernel


def _make_agg_kernel(n_nodes, n_edges_pad, d_half):
    e_tile = n_edges_pad // NS
    n_chunks = e_tile // CB
    r_tile = n_nodes // NS
    zb = 125
    assert r_tile % zb == 0 and zb <= CB

    @functools.partial(
        pl.kernel,
        out_type=[
            jax.ShapeDtypeStruct((n_nodes, d_half), jnp.float32),
            jax.ShapeDtypeStruct((n_nodes, d_half), jnp.float32),
        ],
        mesh=_sc_mesh(),
        scratch_types=[
            pltpu.VMEM((CB,), jnp.int32),
            pltpu.VMEM((CB,), jnp.int32),
            pltpu.VMEM((CB,), jnp.float32),
            pltpu.VMEM((CB,), jnp.float32),
            pltpu.VMEM((CB, d_half), jnp.float32),
            pltpu.VMEM((n_nodes,), jnp.float32),
            pltpu.VMEM_SHARED((n_nodes, d_half), jnp.float32),
        ],
    )
    def agg_kernel(xa_hbm, xb_hbm, src_hbm, dst_hbm, ew_hbm, dis_hbm,
                   oa_hbm, ob_hbm,
                   src_v, dst_v, ew_v, nrm_v, rows_v, dis_v, acc_sh):
        c = lax.axis_index("c")
        s = lax.axis_index("s")
        pltpu.sync_copy(dis_hbm, dis_v)
        _zero_rows(rows_v, CB, d_half)
        for k in range(r_tile // zb):
            pltpu.sync_copy(
                rows_v.at[pl.ds(0, zb)],
                acc_sh.at[pl.ds(s * r_tile + k * zb, zb)],
            )
        plsc.subcore_barrier()

        base = s * e_tile

        def chunk(k, carry):
            off = base + k * CB
            pltpu.sync_copy(src_hbm.at[pl.ds(off, CB)], src_v)
            pltpu.sync_copy(dst_hbm.at[pl.ds(off, CB)], dst_v)
            pltpu.sync_copy(ew_hbm.at[pl.ds(off, CB)], ew_v)

            @pl.when(c == 0)
            def _():
                pltpu.sync_copy(xa_hbm.at[src_v], rows_v)

            @pl.when(c == 1)
            def _():
                pltpu.sync_copy(xb_hbm.at[src_v], rows_v)

            for g in range(CB // L):
                sg = src_v[pl.ds(g * L, L)]
                dg = dst_v[pl.ds(g * L, L)]
                ewg = ew_v[pl.ds(g * L, L)]
                ng = (plsc.load_gather(dis_v, [sg]) * ewg
                      * plsc.load_gather(dis_v, [dg]))
                nrm_v[pl.ds(g * L, L)] = ng

            def scale_row(i, carry2):
                nspl = plsc.load_gather(
                    nrm_v, [jnp.zeros((L,), jnp.int32) + i])
                for j in range(d_half // L):
                    sl = pl.ds(j * L, L)
                    rows_v[i, sl] = rows_v[i, sl] * nspl
                return carry2

            lax.fori_loop(0, CB, scale_row, 0)
            pltpu.sync_copy(rows_v, acc_sh.at[dst_v], add=True)
            return carry

        lax.fori_loop(0, n_chunks, chunk, 0)
        plsc.subcore_barrier()

        ro = s * r_tile

        @pl.when(c == 0)
        def _():
            pltpu.sync_copy(acc_sh.at[pl.ds(ro, r_tile)],
                            oa_hbm.at[pl.ds(ro, r_tile)])

        @pl.when(c == 1)
        def _():
            pltpu.sync_copy(acc_sh.at[pl.ds(ro, r_tile)],
                            ob_hbm.at[pl.ds(ro, r_tile)])

    return agg_kernel


def _tc1(x, w1t, deg3, rows_blk):
    n, in_ch = x.shape
    hid = w1t.shape[1]
    h2 = hid // 2
    grid = n // rows_blk

    def body(x_ref, w_ref, deg_ref, xa_ref, xb_ref, dis_ref, dinv_ref):
        xw = jnp.dot(x_ref[...], w_ref[...],
                     preferred_element_type=jnp.float32)
        xa_ref[...] = xw[:, :h2]
        xb_ref[...] = xw[:, h2:]
        d = 1.0 + jnp.sum(jnp.sum(deg_ref[...], axis=0), axis=-1)
        d = jnp.maximum(d, 1e-30)
        dis_ref[...] = lax.rsqrt(d)[:, None]
        dinv_ref[...] = (1.0 / d)[:, None]

    return pl.pallas_call(
        body,
        grid=(grid,),
        in_specs=[
            pl.BlockSpec((rows_blk, in_ch), lambda i: (i, 0)),
            pl.BlockSpec((in_ch, hid), lambda i: (0, 0)),
            pl.BlockSpec((NC, rows_blk, L), lambda i: (0, i, 0)),
        ],
        out_specs=[
            pl.BlockSpec((rows_blk, h2), lambda i: (i, 0)),
            pl.BlockSpec((rows_blk, h2), lambda i: (i, 0)),
            pl.BlockSpec((rows_blk, 1), lambda i: (i, 0)),
            pl.BlockSpec((rows_blk, 1), lambda i: (i, 0)),
        ],
        out_shape=[
            jax.ShapeDtypeStruct((n, h2), jnp.float32),
            jax.ShapeDtypeStruct((n, h2), jnp.float32),
            jax.ShapeDtypeStruct((n, 1), jnp.float32),
            jax.ShapeDtypeStruct((n, 1), jnp.float32),
        ],
    )(x, w1t, deg3)


def _tc2(aa, ab, xa, xb, dinv, b1, w2t, rows_blk):
    n, h2 = xa.shape
    hid = 2 * h2
    out_ch = w2t.shape[1]
    o2 = out_ch // 2
    grid = n // rows_blk

    def body(aa_ref, ab_ref, xa_ref, xb_ref, dinv_ref, b1_ref, w_ref,
             oa_ref, ob_ref):
        dv = dinv_ref[...]
        ha = jnp.maximum(aa_ref[...] + dv * xa_ref[...] + b1_ref[0, :h2], 0.0)
        hb = jnp.maximum(ab_ref[...] + dv * xb_ref[...] + b1_ref[0, h2:], 0.0)
        h = jnp.concatenate([ha, hb], axis=1)
        hw = jnp.dot(h, w_ref[...], preferred_element_type=jnp.float32)
        oa_ref[...] = hw[:, :o2]
        ob_ref[...] = hw[:, o2:]

    return pl.pallas_call(
        body,
        grid=(grid,),
        in_specs=[
            pl.BlockSpec((rows_blk, h2), lambda i: (i, 0)),
            pl.BlockSpec((rows_blk, h2), lambda i: (i, 0)),
            pl.BlockSpec((rows_blk, h2), lambda i: (i, 0)),
            pl.BlockSpec((rows_blk, h2), lambda i: (i, 0)),
            pl.BlockSpec((rows_blk, 1), lambda i: (i, 0)),
            pl.BlockSpec((1, hid), lambda i: (0, 0)),
            pl.BlockSpec((hid, out_ch), lambda i: (0, 0)),
        ],
        out_specs=[
            pl.BlockSpec((rows_blk, o2), lambda i: (i, 0)),
            pl.BlockSpec((rows_blk, o2), lambda i: (i, 0)),
        ],
        out_shape=[
            jax.ShapeDtypeStruct((n, o2), jnp.float32),
            jax.ShapeDtypeStruct((n, o2), jnp.float32),
        ],
    )(aa, ab, xa, xb, dinv, b1, w2t)


def _tc3(aa, ab, ha, hb, dinv, b2, rows_blk):
    n, o2 = ha.shape
    out_ch = 2 * o2
    grid = n // rows_blk

    def body(aa_ref, ab_ref, ha_ref, hb_ref, dinv_ref, b2_ref, out_ref):
        dv = dinv_ref[...]
        za = aa_ref[...] + dv * ha_ref[...]
        zb = ab_ref[...] + dv * hb_ref[...]
        z = jnp.concatenate([za, zb], axis=1) + b2_ref[0, :]
        m = jnp.max(z, axis=1, keepdims=True)
        e = z - m
        lse = jnp.log(jnp.sum(jnp.exp(e), axis=1, keepdims=True))
        out_ref[...] = e - lse

    return pl.pallas_call(
        body,
        grid=(grid,),
        in_specs=[
            pl.BlockSpec((rows_blk, o2), lambda i: (i, 0)),
            pl.BlockSpec((rows_blk, o2), lambda i: (i, 0)),
            pl.BlockSpec((rows_blk, o2), lambda i: (i, 0)),
            pl.BlockSpec((rows_blk, o2), lambda i: (i, 0)),
            pl.BlockSpec((rows_blk, 1), lambda i: (i, 0)),
            pl.BlockSpec((1, out_ch), lambda i: (0, 0)),
        ],
        out_specs=pl.BlockSpec((rows_blk, out_ch), lambda i: (i, 0)),
        out_shape=jax.ShapeDtypeStruct((n, out_ch), jnp.float32),
    )(aa, ab, ha, hb, dinv, b2)


@jax.jit
def kernel(x, edge_index, edge_weight, W1, b1, W2, b2):
    n, _ = x.shape
    hid = W1.shape[0]
    out_ch = W2.shape[0]
    e = edge_index.shape[1]

    src = edge_index[0].astype(jnp.int32)
    dst = edge_index[1].astype(jnp.int32)
    ew = edge_weight.astype(jnp.float32)

    ep = ((e + NC * NS * CB - 1) // (NC * NS * CB)) * (NC * NS * CB)
    pad = ep - e
    if pad:
        src = jnp.concatenate([src, jnp.zeros((pad,), jnp.int32)])
        dst = jnp.concatenate([dst, jnp.zeros((pad,), jnp.int32)])
        ew = jnp.concatenate([ew, jnp.zeros((pad,), jnp.float32)])

    rows_blk = 500

    deg3 = _make_deg_kernel(n, ep)(dst, ew)
    xa, xb, dis2, dinv2 = _tc1(x, W1.T, deg3, rows_blk)
    dis = dis2.reshape(n)

    aa, ab = _make_agg_kernel(n, ep, hid // 2)(xa, xb, src, dst, ew, dis)
    hwa, hwb = _tc2(aa, ab, xa, xb, dinv2, b1.reshape(1, hid), W2.T, rows_blk)

    ga, gb = _make_agg_kernel(n, ep, out_ch // 2)(hwa, hwb, src, dst, ew, dis)
    return _tc3(ga, gb, hwa, hwb, dinv2, b2.reshape(1, out_ch), rows_blk)


# speedup vs baseline: 5.1999x; 5.1999x over previous
import functools

import jax
import jax.numpy as jnp
from jax import lax
from jax.experimental import pallas as pl
from jax.experimental.pallas import tpu as pltpu
from jax.experimental.pallas import tpu_sc as plsc

NC = 2
NS = 16
L = 16
CB = 128


def _sc_mesh():
    return plsc.VectorSubcoreMesh(core_axis_name="c", subcore_axis_name="s")


_SC_PARAMS = dict(
    mesh=_sc_mesh(),
    compiler_params=pltpu.CompilerParams(needs_layout_passes=False),
)


def _zero_rows(ref, n_rows, n_cols):
    zeros = jnp.zeros((L,), jnp.float32)

    def body(i, carry):
        for j in range(n_cols // L):
            ref[i, pl.ds(j * L, L)] = zeros
        return carry

    lax.fori_loop(0, n_rows, body, 0)


def _make_deg_kernel(n_pad, n_edges_pad):
    e_worker = n_edges_pad // (NC * NS)
    n_chunks = e_worker // CB

    @functools.partial(
        pl.kernel,
        out_type=jax.ShapeDtypeStruct((NC, NS, n_pad), jnp.float32),
        scratch_types=[
            pltpu.VMEM((CB,), jnp.int32),
            pltpu.VMEM((CB,), jnp.float32),
            pltpu.VMEM((n_pad,), jnp.float32),
        ],
        **_SC_PARAMS,
    )
    def deg_kernel(dst_hbm, ew_hbm, out_hbm, dst_v, ew_v, deg_v):
        c = lax.axis_index("c")
        s = lax.axis_index("s")
        zeros = jnp.zeros((L,), jnp.float32)

        def zbody(i, carry):
            deg_v[pl.ds(i * L, L)] = zeros
            return carry

        lax.fori_loop(0, n_pad // L, zbody, 0)

        base = (c * NS + s) * e_worker
        lane = lax.iota(jnp.int32, L)

        def chunk(k, carry):
            off = base + k * CB
            pltpu.sync_copy(dst_hbm.at[pl.ds(off, CB)], dst_v)
            pltpu.sync_copy(ew_hbm.at[pl.ds(off, CB)], ew_v)
            for g in range(CB // L):
                dg = dst_v[pl.ds(g * L, L)]
                ewg = ew_v[pl.ds(g * L, L)]
                for l in range(L):
                    plsc.addupdate_scatter(deg_v, [dg], ewg,
                                           mask=lane == l)
            return carry

        lax.fori_loop(0, n_chunks, chunk, 0)
        pltpu.sync_copy(deg_v, out_hbm.at[c].at[s])

    return deg_kernel


def _agg_chunk_body(src_v, dst_v, ew_v, nrm_v, rows_v, dis_v, acc_sh, d):
    for g in range(CB // L):
        sg = src_v[pl.ds(g * L, L)]
        dg = dst_v[pl.ds(g * L, L)]
        ewg = ew_v[pl.ds(g * L, L)]
        ng = (plsc.load_gather(dis_v, [sg]) * ewg
              * plsc.load_gather(dis_v, [dg]))
        nrm_v[pl.ds(g * L, L)] = ng

    def scale_row(i, carry):
        nspl = plsc.load_gather(nrm_v, [jnp.zeros((L,), jnp.int32) + i])
        for j in range(d // L):
            sl = pl.ds(j * L, L)
            rows_v[i, sl] = rows_v[i, sl] * nspl
        return carry

    lax.fori_loop(0, CB, scale_row, 0)
    pltpu.sync_copy(rows_v, acc_sh.at[dst_v], add=True)


def _make_agg_feat_split(n_nodes, n_pad, n_edges_pad, d_half):
    e_tile = n_edges_pad // NS
    n_chunks = e_tile // CB
    r_tile = n_pad // NS
    assert r_tile % CB == 0 and d_half % L == 0

    @functools.partial(
        pl.kernel,
        out_type=[
            jax.ShapeDtypeStruct((n_pad, d_half), jnp.float32),
            jax.ShapeDtypeStruct((n_pad, d_half), jnp.float32),
        ],
        scratch_types=[
            pltpu.VMEM((CB,), jnp.int32),
            pltpu.VMEM((CB,), jnp.int32),
            pltpu.VMEM((CB,), jnp.float32),
            pltpu.VMEM((CB,), jnp.float32),
            pltpu.VMEM((CB, d_half), jnp.float32),
            pltpu.VMEM((n_nodes,), jnp.float32),
            pltpu.VMEM_SHARED((n_pad, d_half), jnp.float32),
        ],
        **_SC_PARAMS,
    )
    def agg_kernel(xa_hbm, xb_hbm, src_hbm, dst_hbm, ew_hbm, dis_hbm,
                   oa_hbm, ob_hbm,
                   src_v, dst_v, ew_v, nrm_v, rows_v, dis_v, acc_sh):
        c = lax.axis_index("c")
        s = lax.axis_index("s")
        pltpu.sync_copy(dis_hbm, dis_v)
        _zero_rows(rows_v, CB, d_half)
        for k in range(r_tile // CB):
            pltpu.sync_copy(rows_v,
                            acc_sh.at[pl.ds(s * r_tile + k * CB, CB)])
        plsc.subcore_barrier()

        base = s * e_tile

        def chunk(k, carry):
            off = base + k * CB
            pltpu.sync_copy(src_hbm.at[pl.ds(off, CB)], src_v)
            pltpu.sync_copy(dst_hbm.at[pl.ds(off, CB)], dst_v)
            pltpu.sync_copy(ew_hbm.at[pl.ds(off, CB)], ew_v)

            @pl.when(c == 0)
            def _():
                pltpu.sync_copy(xa_hbm.at[src_v], rows_v)

            @pl.when(c == 1)
            def _():
                pltpu.sync_copy(xb_hbm.at[src_v], rows_v)

            _agg_chunk_body(src_v, dst_v, ew_v, nrm_v, rows_v, dis_v,
                            acc_sh, d_half)
            return carry

        lax.fori_loop(0, n_chunks, chunk, 0)
        plsc.subcore_barrier()

        ro = s * r_tile

        @pl.when(c == 0)
        def _():
            pltpu.sync_copy(acc_sh.at[pl.ds(ro, r_tile)],
                            oa_hbm.at[pl.ds(ro, r_tile)])

        @pl.when(c == 1)
        def _():
            pltpu.sync_copy(acc_sh.at[pl.ds(ro, r_tile)],
                            ob_hbm.at[pl.ds(ro, r_tile)])

    return agg_kernel


def _make_agg_edge_split(n_nodes, n_pad, n_edges_pad, d):
    e_core = n_edges_pad // NC
    e_tile = e_core // NS
    n_chunks = e_tile // CB
    r_tile = n_pad // NS
    assert r_tile % CB == 0 and d % L == 0

    @functools.partial(
        pl.kernel,
        out_type=jax.ShapeDtypeStruct((NC, n_pad, d), jnp.float32),
        scratch_types=[
            pltpu.VMEM((CB,), jnp.int32),
            pltpu.VMEM((CB,), jnp.int32),
            pltpu.VMEM((CB,), jnp.float32),
            pltpu.VMEM((CB,), jnp.float32),
            pltpu.VMEM((CB, d), jnp.float32),
            pltpu.VMEM((n_nodes,), jnp.float32),
            pltpu.VMEM_SHARED((n_pad, d), jnp.float32),
        ],
        **_SC_PARAMS,
    )
    def agg_kernel(xw_hbm, src_hbm, dst_hbm, ew_hbm, dis_hbm, out_hbm,
                   src_v, dst_v, ew_v, nrm_v, rows_v, dis_v, acc_sh):
        c = lax.axis_index("c")
        s = lax.axis_index("s")
        pltpu.sync_copy(dis_hbm, dis_v)
        _zero_rows(rows_v, CB, d)
        for k in range(r_tile // CB):
            pltpu.sync_copy(rows_v,
                            acc_sh.at[pl.ds(s * r_tile + k * CB, CB)])
        plsc.subcore_barrier()

        base = c * e_core + s * e_tile

        def chunk(k, carry):
            off = base + k * CB
            pltpu.sync_copy(src_hbm.at[pl.ds(off, CB)], src_v)
            pltpu.sync_copy(dst_hbm.at[pl.ds(off, CB)], dst_v)
            pltpu.sync_copy(ew_hbm.at[pl.ds(off, CB)], ew_v)
            pltpu.sync_copy(xw_hbm.at[src_v], rows_v)
            _agg_chunk_body(src_v, dst_v, ew_v, nrm_v, rows_v, dis_v,
                            acc_sh, d)
            return carry

        lax.fori_loop(0, n_chunks, chunk, 0)
        plsc.subcore_barrier()

        ro = s * r_tile
        pltpu.sync_copy(acc_sh.at[pl.ds(ro, r_tile)],
                        out_hbm.at[c].at[pl.ds(ro, r_tile)])

    return agg_kernel


def _tc_degnorm(deg32):
    w, n_pad = deg32.shape

    def body(deg_ref, dis_ref, dinv_ref):
        d = 1.0 + jnp.sum(deg_ref[...], axis=0, keepdims=True)
        d = jnp.maximum(d, 1e-30)
        dis_ref[...] = lax.rsqrt(d)
        dinv_ref[...] = 1.0 / d

    return pl.pallas_call(
        body,
        out_shape=[
            jax.ShapeDtypeStruct((1, n_pad), jnp.float32),
            jax.ShapeDtypeStruct((1, n_pad), jnp.float32),
        ],
    )(deg32)


def _tc1(x, w1t, rows_blk):
    n, in_ch = x.shape
    hid = w1t.shape[1]
    h2 = hid // 2
    grid = n // rows_blk

    def body(x_ref, w_ref, xa_ref, xb_ref):
        xw = jnp.dot(x_ref[...], w_ref[...],
                     preferred_element_type=jnp.float32)
        xa_ref[...] = xw[:, :h2]
        xb_ref[...] = xw[:, h2:]

    return pl.pallas_call(
        body,
        grid=(grid,),
        in_specs=[
            pl.BlockSpec((rows_blk, in_ch), lambda i: (i, 0)),
            pl.BlockSpec((in_ch, hid), lambda i: (0, 0)),
        ],
        out_specs=[
            pl.BlockSpec((rows_blk, h2), lambda i: (i, 0)),
            pl.BlockSpec((rows_blk, h2), lambda i: (i, 0)),
        ],
        out_shape=[
            jax.ShapeDtypeStruct((n, h2), jnp.float32),
            jax.ShapeDtypeStruct((n, h2), jnp.float32),
        ],
    )(x, w1t)


def _tc2(aa, ab, xa, xb, dinv, b1, w2t, d_pad, rows_blk):
    n, h2 = xa.shape
    hid = 2 * h2
    out_ch = w2t.shape[1]
    grid = n // rows_blk

    def body(aa_ref, ab_ref, xa_ref, xb_ref, dinv_ref, b1_ref, w_ref,
             hw_ref):
        dv = dinv_ref[...]
        ha = jnp.maximum(aa_ref[...] + dv * xa_ref[...] + b1_ref[0, :h2], 0.0)
        hb = jnp.maximum(ab_ref[...] + dv * xb_ref[...] + b1_ref[0, h2:], 0.0)
        h = jnp.concatenate([ha, hb], axis=1)
        hw = jnp.dot(h, w_ref[...], preferred_element_type=jnp.float32)
        hw_ref[...] = jnp.concatenate(
            [hw, jnp.zeros((rows_blk, d_pad - out_ch), jnp.float32)], axis=1)

    return pl.pallas_call(
        body,
        grid=(grid,),
        in_specs=[
            pl.BlockSpec((rows_blk, h2), lambda i: (i, 0)),
            pl.BlockSpec((rows_blk, h2), lambda i: (i, 0)),
            pl.BlockSpec((rows_blk, h2), lambda i: (i, 0)),
            pl.BlockSpec((rows_blk, h2), lambda i: (i, 0)),
            pl.BlockSpec((rows_blk, 1), lambda i: (i, 0)),
            pl.BlockSpec((1, hid), lambda i: (0, 0)),
            pl.BlockSpec((hid, out_ch), lambda i: (0, 0)),
        ],
        out_specs=pl.BlockSpec((rows_blk, d_pad), lambda i: (i, 0)),
        out_shape=jax.ShapeDtypeStruct((n, d_pad), jnp.float32),
    )(aa, ab, xa, xb, dinv, b1, w2t)


def _tc3(g2, hw, dinv, b2, out_ch, rows_blk):
    n, d_pad = hw.shape
    grid = n // rows_blk

    def body(g_ref, hw_ref, dinv_ref, b2_ref, out_ref):
        gsum = jnp.sum(g_ref[...], axis=0)
        zf = gsum + dinv_ref[...] * hw_ref[...]
        z = zf[:, :out_ch] + b2_ref[0, :]
        m = jnp.max(z, axis=1, keepdims=True)
        e = z - m
        lse = jnp.log(jnp.sum(jnp.exp(e), axis=1, keepdims=True))
        out_ref[...] = e - lse

    return pl.pallas_call(
        body,
        grid=(grid,),
        in_specs=[
            pl.BlockSpec((NC, rows_blk, d_pad), lambda i: (0, i, 0)),
            pl.BlockSpec((rows_blk, d_pad), lambda i: (i, 0)),
            pl.BlockSpec((rows_blk, 1), lambda i: (i, 0)),
            pl.BlockSpec((1, out_ch), lambda i: (0, 0)),
        ],
        out_specs=pl.BlockSpec((rows_blk, out_ch), lambda i: (i, 0)),
        out_shape=jax.ShapeDtypeStruct((n, out_ch), jnp.float32),
    )(g2, hw, dinv, b2)


@jax.jit
def kernel(x, edge_index, edge_weight, W1, b1, W2, b2):
    n, _ = x.shape
    hid = W1.shape[0]
    out_ch = W2.shape[0]
    e = edge_index.shape[1]
    d_pad = 128

    src = edge_index[0].astype(jnp.int32)
    dst = edge_index[1].astype(jnp.int32)
    ew = edge_weight.astype(jnp.float32)

    ep = ((e + NC * NS * CB - 1) // (NC * NS * CB)) * (NC * NS * CB)
    pad = ep - e
    if pad:
        src = jnp.concatenate([src, jnp.zeros((pad,), jnp.int32)])
        dst = jnp.concatenate([dst, jnp.zeros((pad,), jnp.int32)])
        ew = jnp.concatenate([ew, jnp.zeros((pad,), jnp.float32)])

    rows_blk = 1000
    n_pad = ((n + NS * CB - 1) // (NS * CB)) * (NS * CB)

    deg32 = _make_deg_kernel(n_pad, ep)(dst, ew)
    dis_r, dinv_r = _tc_degnorm(deg32.reshape(NC * NS, n_pad))
    dis = dis_r.reshape(n_pad)[:n]
    dinv = dinv_r.reshape(n_pad)[:n].reshape(n, 1)

    xa, xb = _tc1(x, W1.T, rows_blk)
    aa, ab = _make_agg_feat_split(n, n_pad, ep, hid // 2)(
        xa, xb, src, dst, ew, dis)
    hw = _tc2(aa, ab, xa, xb, dinv, b1.reshape(1, hid), W2.T, d_pad, rows_blk)
    g2 = _make_agg_edge_split(n, n_pad, ep, d_pad)(hw, src, dst, ew, dis)
    return _tc3(g2, hw, dinv, b2.reshape(1, out_ch), out_ch, rows_blk)


# speedup vs baseline: 7.1756x; 1.3799x over previous
import functools

import jax
import jax.numpy as jnp
from jax import lax
from jax.experimental import pallas as pl
from jax.experimental.pallas import tpu as pltpu
from jax.experimental.pallas import tpu_sc as plsc

NC = 2
NS = 16
L = 16
CB = 128


def _sc_mesh():
    return plsc.VectorSubcoreMesh(core_axis_name="c", subcore_axis_name="s")


_SC_PARAMS = dict(
    mesh=_sc_mesh(),
    compiler_params=pltpu.CompilerParams(needs_layout_passes=False),
)


def _zero_rows(ref, n_rows, n_cols):
    zeros = jnp.zeros((L,), jnp.float32)

    def body(i, carry):
        for j in range(n_cols // L):
            ref[i, pl.ds(j * L, L)] = zeros
        return carry

    lax.fori_loop(0, n_rows, body, 0)


def _make_deg_kernel(n_pad, n_edges_pad):
    rpt = n_edges_pad // CB // (NC * NS)

    @functools.partial(
        pl.kernel,
        out_type=jax.ShapeDtypeStruct((NC, NS, n_pad), jnp.float32),
        scratch_types=[
            pltpu.VMEM((rpt, CB), jnp.int32),
            pltpu.VMEM((rpt, CB), jnp.float32),
            pltpu.VMEM((n_pad,), jnp.float32),
        ],
        **_SC_PARAMS,
    )
    def deg_kernel(dst_hbm, ew_hbm, out_hbm, dst_sup, ew_sup, deg_v):
        c = lax.axis_index("c")
        s = lax.axis_index("s")
        base_row = (c * NS + s) * rpt
        pltpu.sync_copy(dst_hbm.at[pl.ds(base_row, rpt)], dst_sup)
        pltpu.sync_copy(ew_hbm.at[pl.ds(base_row, rpt)], ew_sup)
        zeros = jnp.zeros((L,), jnp.float32)

        def zbody(i, carry):
            deg_v[pl.ds(i * L, L)] = zeros
            return carry

        lax.fori_loop(0, n_pad // L, zbody, 0)
        lane = lax.iota(jnp.int32, L)

        def chunk(k, carry):
            for g in range(CB // L):
                dg = dst_sup[k, pl.ds(g * L, L)]
                ewg = ew_sup[k, pl.ds(g * L, L)]
                for l in range(L):
                    plsc.addupdate_scatter(deg_v, [dg], ewg,
                                           mask=lane == l)
            return carry

        lax.fori_loop(0, rpt, chunk, 0)
        pltpu.sync_copy(deg_v, out_hbm.at[c].at[s])

    return deg_kernel


SUP = 8


def _agg_pipeline(s, rpt, d, r_tile,
                  src_sup, dst_sup, ew_sup, nrm_v, rows, dis_v, acc_sh,
                  sem_g, sem_s,
                  dis_hbm, src_hbm, dst_hbm, ew_hbm, rep_tbl,
                  base_row, start_gather):
    pltpu.sync_copy(dis_hbm, dis_v)

    _zero_rows(rows[1], CB, d)
    for k in range(r_tile // CB):
        pltpu.sync_copy(rows[1], acc_sh.at[pl.ds(s * r_tile + k * CB, CB)])
    plsc.subcore_barrier()

    def wait_gather(b):
        pltpu.make_async_copy(rep_tbl.at[src_sup.at[0]], rows[b],
                              sem_g[b]).wait()

    def wait_scatter(b):
        pltpu.make_async_copy(rows[b], acc_sh.at[dst_sup.at[0]],
                              sem_s[b]).wait()

    def super_body(sb, carry):
        row0 = base_row + sb * SUP
        pltpu.sync_copy(src_hbm.at[pl.ds(row0, SUP)], src_sup)
        pltpu.sync_copy(dst_hbm.at[pl.ds(row0, SUP)], dst_sup)
        pltpu.sync_copy(ew_hbm.at[pl.ds(row0, SUP)], ew_sup)
        start_gather(0, 0)

        def body(jj, carry2):
            for b in range(2):
                j = 2 * jj + b

                @pl.when(j + 1 < SUP)
                def _():
                    @pl.when(j >= 1)
                    def _():
                        wait_scatter(1 - b)

                    start_gather(j + 1, 1 - b)

                wait_gather(b)
                for g in range(CB // L):
                    sg = src_sup[j, pl.ds(g * L, L)]
                    dg = dst_sup[j, pl.ds(g * L, L)]
                    ewg = ew_sup[j, pl.ds(g * L, L)]
                    ng = (plsc.load_gather(dis_v, [sg]) * ewg
                          * plsc.load_gather(dis_v, [dg]))
                    nrm_v[pl.ds(g * L, L)] = ng

                rb = rows[b]

                def scale_row(i, carry3):
                    nspl = plsc.load_gather(
                        nrm_v, [jnp.zeros((L,), jnp.int32) + i])
                    for jv in range(d // L):
                        sl = pl.ds(jv * L, L)
                        rb[i, sl] = rb[i, sl] * nspl
                    return carry3

                lax.fori_loop(0, CB, scale_row, 0)
                pltpu.async_copy(rows[b], acc_sh.at[dst_sup.at[j]],
                                 sem_s[b], add=True)
            return carry2

        lax.fori_loop(0, SUP // 2, body, 0)
        wait_scatter(0)
        wait_scatter(1)
        return carry

    lax.fori_loop(0, rpt // SUP, super_body, 0)
    plsc.subcore_barrier()


def _make_agg_feat_split(n_nodes, n_pad, n_edges_pad, d_half):
    rpt = n_edges_pad // CB // NS
    r_tile = n_pad // NS
    assert r_tile % CB == 0 and d_half % L == 0 and rpt % SUP == 0

    @functools.partial(
        pl.kernel,
        out_type=[
            jax.ShapeDtypeStruct((n_pad, d_half), jnp.float32),
            jax.ShapeDtypeStruct((n_pad, d_half), jnp.float32),
        ],
        scratch_types=[
            pltpu.VMEM((SUP, CB), jnp.int32),
            pltpu.VMEM((SUP, CB), jnp.int32),
            pltpu.VMEM((SUP, CB), jnp.float32),
            pltpu.VMEM((CB,), jnp.float32),
            pltpu.VMEM((CB, d_half), jnp.float32),
            pltpu.VMEM((CB, d_half), jnp.float32),
            pltpu.VMEM((n_nodes,), jnp.float32),
            pltpu.VMEM_SHARED((n_pad, d_half), jnp.float32),
            pltpu.SemaphoreType.DMA,
            pltpu.SemaphoreType.DMA,
            pltpu.SemaphoreType.DMA,
            pltpu.SemaphoreType.DMA,
        ],
        **_SC_PARAMS,
    )
    def agg_kernel(xa_hbm, xb_hbm, src_hbm, dst_hbm, ew_hbm, dis_hbm,
                   oa_hbm, ob_hbm,
                   src_sup, dst_sup, ew_sup, nrm_v, rows0, rows1, dis_v,
                   acc_sh, sg0, sg1, ss0, ss1):
        c = lax.axis_index("c")
        s = lax.axis_index("s")
        rows = (rows0, rows1)
        sem_g = (sg0, sg1)
        sem_s = (ss0, ss1)

        def start_gather(j, b):
            @pl.when(c == 0)
            def _():
                pltpu.async_copy(xa_hbm.at[src_sup.at[j]], rows[b],
                                 sem_g[b])

            @pl.when(c == 1)
            def _():
                pltpu.async_copy(xb_hbm.at[src_sup.at[j]], rows[b],
                                 sem_g[b])

        _agg_pipeline(s, rpt, d_half, r_tile,
                      src_sup, dst_sup, ew_sup, nrm_v, rows, dis_v, acc_sh,
                      sem_g, sem_s,
                      dis_hbm, src_hbm, dst_hbm, ew_hbm, xa_hbm,
                      s * rpt, start_gather)

        ro = s * r_tile

        @pl.when(c == 0)
        def _():
            pltpu.sync_copy(acc_sh.at[pl.ds(ro, r_tile)],
                            oa_hbm.at[pl.ds(ro, r_tile)])

        @pl.when(c == 1)
        def _():
            pltpu.sync_copy(acc_sh.at[pl.ds(ro, r_tile)],
                            ob_hbm.at[pl.ds(ro, r_tile)])

    return agg_kernel


def _make_agg_edge_split(n_nodes, n_pad, n_edges_pad, d):
    rpt = n_edges_pad // CB // (NC * NS)
    r_tile = n_pad // NS
    assert r_tile % CB == 0 and d % L == 0 and rpt % SUP == 0

    @functools.partial(
        pl.kernel,
        out_type=jax.ShapeDtypeStruct((NC, n_pad, d), jnp.float32),
        scratch_types=[
            pltpu.VMEM((SUP, CB), jnp.int32),
            pltpu.VMEM((SUP, CB), jnp.int32),
            pltpu.VMEM((SUP, CB), jnp.float32),
            pltpu.VMEM((CB,), jnp.float32),
            pltpu.VMEM((CB, d), jnp.float32),
            pltpu.VMEM((CB, d), jnp.float32),
            pltpu.VMEM((n_nodes,), jnp.float32),
            pltpu.VMEM_SHARED((n_pad, d), jnp.float32),
            pltpu.SemaphoreType.DMA,
            pltpu.SemaphoreType.DMA,
            pltpu.SemaphoreType.DMA,
            pltpu.SemaphoreType.DMA,
        ],
        **_SC_PARAMS,
    )
    def agg_kernel(xw_hbm, src_hbm, dst_hbm, ew_hbm, dis_hbm, out_hbm,
                   src_sup, dst_sup, ew_sup, nrm_v, rows0, rows1, dis_v,
                   acc_sh, sg0, sg1, ss0, ss1):
        c = lax.axis_index("c")
        s = lax.axis_index("s")
        rows = (rows0, rows1)
        sem_g = (sg0, sg1)
        sem_s = (ss0, ss1)

        def start_gather(j, b):
            pltpu.async_copy(xw_hbm.at[src_sup.at[j]], rows[b], sem_g[b])

        _agg_pipeline(s, rpt, d, r_tile,
                      src_sup, dst_sup, ew_sup, nrm_v, rows, dis_v, acc_sh,
                      sem_g, sem_s,
                      dis_hbm, src_hbm, dst_hbm, ew_hbm, xw_hbm,
                      (c * NS + s) * rpt, start_gather)

        ro = s * r_tile
        pltpu.sync_copy(acc_sh.at[pl.ds(ro, r_tile)],
                        out_hbm.at[c].at[pl.ds(ro, r_tile)])

    return agg_kernel


def _tc_degnorm(deg32):
    w, n_pad = deg32.shape

    def body(deg_ref, dis_ref, dinv_ref):
        d = 1.0 + jnp.sum(deg_ref[...], axis=0, keepdims=True)
        d = jnp.maximum(d, 1e-30)
        dis_ref[...] = lax.rsqrt(d)
        dinv_ref[...] = 1.0 / d

    return pl.pallas_call(
        body,
        out_shape=[
            jax.ShapeDtypeStruct((1, n_pad), jnp.float32),
            jax.ShapeDtypeStruct((1, n_pad), jnp.float32),
        ],
    )(deg32)


def _tc1(x, w1t, rows_blk):
    n, in_ch = x.shape
    hid = w1t.shape[1]
    h2 = hid // 2
    grid = n // rows_blk

    def body(x_ref, w_ref, xa_ref, xb_ref):
        xw = jnp.dot(x_ref[...], w_ref[...],
                     preferred_element_type=jnp.float32)
        xa_ref[...] = xw[:, :h2]
        xb_ref[...] = xw[:, h2:]

    return pl.pallas_call(
        body,
        grid=(grid,),
        in_specs=[
            pl.BlockSpec((rows_blk, in_ch), lambda i: (i, 0)),
            pl.BlockSpec((in_ch, hid), lambda i: (0, 0)),
        ],
        out_specs=[
            pl.BlockSpec((rows_blk, h2), lambda i: (i, 0)),
            pl.BlockSpec((rows_blk, h2), lambda i: (i, 0)),
        ],
        out_shape=[
            jax.ShapeDtypeStruct((n, h2), jnp.float32),
            jax.ShapeDtypeStruct((n, h2), jnp.float32),
        ],
    )(x, w1t)


def _tc2(aa, ab, xa, xb, dinv, b1, w2t, d_pad, rows_blk):
    n, h2 = xa.shape
    hid = 2 * h2
    out_ch = w2t.shape[1]
    grid = n // rows_blk

    def body(aa_ref, ab_ref, xa_ref, xb_ref, dinv_ref, b1_ref, w_ref,
             hw_ref):
        dv = dinv_ref[...]
        ha = jnp.maximum(aa_ref[...] + dv * xa_ref[...] + b1_ref[0, :h2], 0.0)
        hb = jnp.maximum(ab_ref[...] + dv * xb_ref[...] + b1_ref[0, h2:], 0.0)
        h = jnp.concatenate([ha, hb], axis=1)
        hw = jnp.dot(h, w_ref[...], preferred_element_type=jnp.float32)
        hw_ref[...] = jnp.concatenate(
            [hw, jnp.zeros((rows_blk, d_pad - out_ch), jnp.float32)], axis=1)

    return pl.pallas_call(
        body,
        grid=(grid,),
        in_specs=[
            pl.BlockSpec((rows_blk, h2), lambda i: (i, 0)),
            pl.BlockSpec((rows_blk, h2), lambda i: (i, 0)),
            pl.BlockSpec((rows_blk, h2), lambda i: (i, 0)),
            pl.BlockSpec((rows_blk, h2), lambda i: (i, 0)),
            pl.BlockSpec((rows_blk, 1), lambda i: (i, 0)),
            pl.BlockSpec((1, hid), lambda i: (0, 0)),
            pl.BlockSpec((hid, out_ch), lambda i: (0, 0)),
        ],
        out_specs=pl.BlockSpec((rows_blk, d_pad), lambda i: (i, 0)),
        out_shape=jax.ShapeDtypeStruct((n, d_pad), jnp.float32),
    )(aa, ab, xa, xb, dinv, b1, w2t)


def _tc3(g2, hw, dinv, b2, out_ch, rows_blk):
    n, d_pad = hw.shape
    grid = n // rows_blk

    def body(g_ref, hw_ref, dinv_ref, b2_ref, out_ref):
        gsum = jnp.sum(g_ref[...], axis=0)
        zf = gsum + dinv_ref[...] * hw_ref[...]
        z = zf[:, :out_ch] + b2_ref[0, :]
        m = jnp.max(z, axis=1, keepdims=True)
        e = z - m
        lse = jnp.log(jnp.sum(jnp.exp(e), axis=1, keepdims=True))
        out_ref[...] = e - lse

    return pl.pallas_call(
        body,
        grid=(grid,),
        in_specs=[
            pl.BlockSpec((NC, rows_blk, d_pad), lambda i: (0, i, 0)),
            pl.BlockSpec((rows_blk, d_pad), lambda i: (i, 0)),
            pl.BlockSpec((rows_blk, 1), lambda i: (i, 0)),
            pl.BlockSpec((1, out_ch), lambda i: (0, 0)),
        ],
        out_specs=pl.BlockSpec((rows_blk, out_ch), lambda i: (i, 0)),
        out_shape=jax.ShapeDtypeStruct((n, out_ch), jnp.float32),
    )(g2, hw, dinv, b2)


@jax.jit
def kernel(x, edge_index, edge_weight, W1, b1, W2, b2):
    n, _ = x.shape
    hid = W1.shape[0]
    out_ch = W2.shape[0]
    e = edge_index.shape[1]
    d_pad = 128

    src = edge_index[0].astype(jnp.int32)
    dst = edge_index[1].astype(jnp.int32)
    ew = edge_weight.astype(jnp.float32)

    ep = ((e + NC * NS * CB - 1) // (NC * NS * CB)) * (NC * NS * CB)
    pad = ep - e
    if pad:
        src = jnp.concatenate([src, jnp.zeros((pad,), jnp.int32)])
        dst = jnp.concatenate([dst, jnp.zeros((pad,), jnp.int32)])
        ew = jnp.concatenate([ew, jnp.zeros((pad,), jnp.float32)])
    src2 = src.reshape(ep // CB, CB)
    dst2 = dst.reshape(ep // CB, CB)
    ew2 = ew.reshape(ep // CB, CB)

    rows_blk = 1000
    n_pad = ((n + NS * CB - 1) // (NS * CB)) * (NS * CB)

    deg32 = _make_deg_kernel(n_pad, ep)(dst2, ew2)
    dis_r, dinv_r = _tc_degnorm(deg32.reshape(NC * NS, n_pad))
    dis = dis_r.reshape(n_pad)[:n]
    dinv = dinv_r.reshape(n_pad)[:n].reshape(n, 1)

    xa, xb = _tc1(x, W1.T, rows_blk)
    aa, ab = _make_agg_feat_split(n, n_pad, ep, hid // 2)(
        xa, xb, src2, dst2, ew2, dis)
    hw = _tc2(aa, ab, xa, xb, dinv, b1.reshape(1, hid), W2.T, d_pad, rows_blk)
    g2 = _make_agg_edge_split(n, n_pad, ep, d_pad)(
        hw, src2, dst2, ew2, dis)
    return _tc3(g2, hw, dinv, b2.reshape(1, out_ch), out_ch, rows_blk)


# speedup vs baseline: 8.3170x; 1.1591x over previous
import functools

import jax
import jax.numpy as jnp
from jax import lax
from jax.experimental import pallas as pl
from jax.experimental.pallas import tpu as pltpu
from jax.experimental.pallas import tpu_sc as plsc

NC = 2
NS = 16
L = 16
CB = 128


def _sc_mesh():
    return plsc.VectorSubcoreMesh(core_axis_name="c", subcore_axis_name="s")


_SC_PARAMS = dict(
    mesh=_sc_mesh(),
    compiler_params=pltpu.CompilerParams(needs_layout_passes=False),
)


def _zero_rows(ref, n_rows, n_cols):
    zeros = jnp.zeros((L,), jnp.float32)

    def body(i, carry):
        for j in range(n_cols // L):
            ref[i, pl.ds(j * L, L)] = zeros
        return carry

    lax.fori_loop(0, n_rows, body, 0)


def _make_deg_kernel(n_pad, n_edges_pad):
    rpt = n_edges_pad // CB // (NC * NS)

    @functools.partial(
        pl.kernel,
        out_type=jax.ShapeDtypeStruct((NC, NS, n_pad), jnp.float32),
        scratch_types=[
            pltpu.VMEM((rpt, CB), jnp.int32),
            pltpu.VMEM((rpt, CB), jnp.float32),
            pltpu.VMEM((n_pad,), jnp.float32),
        ],
        **_SC_PARAMS,
    )
    def deg_kernel(dst_hbm, ew_hbm, out_hbm, dst_sup, ew_sup, deg_v):
        c = lax.axis_index("c")
        s = lax.axis_index("s")
        base_row = (c * NS + s) * rpt
        pltpu.sync_copy(dst_hbm.at[pl.ds(base_row, rpt)], dst_sup)
        pltpu.sync_copy(ew_hbm.at[pl.ds(base_row, rpt)], ew_sup)
        zeros = jnp.zeros((L,), jnp.float32)

        def zbody(i, carry):
            deg_v[pl.ds(i * L, L)] = zeros
            return carry

        lax.fori_loop(0, n_pad // L, zbody, 0)
        lane = lax.iota(jnp.int32, L)

        def chunk(k, carry):
            for g in range(CB // L):
                dg = dst_sup[k, pl.ds(g * L, L)]
                ewg = ew_sup[k, pl.ds(g * L, L)]
                for l in range(L):
                    plsc.addupdate_scatter(deg_v, [dg], ewg,
                                           mask=lane == l)
            return carry

        lax.fori_loop(0, rpt, chunk, 0)
        pltpu.sync_copy(deg_v, out_hbm.at[c].at[s])

    return deg_kernel


SUP = 8


def _agg_pipeline(s, rpt, d, r_tile,
                  src_sup, dst_sup, ew_sup, nrm_v, rows, dis_v, acc_sh,
                  sem_g, sem_s,
                  dis_hbm, src_hbm, dst_hbm, ew_hbm, rep_tbl,
                  base_row, start_gather):
    pltpu.sync_copy(dis_hbm, dis_v)

    _zero_rows(rows[1], CB, d)
    for k in range(r_tile // CB):
        pltpu.sync_copy(rows[1], acc_sh.at[pl.ds(s * r_tile + k * CB, CB)])
    plsc.subcore_barrier()

    def wait_gather(b):
        pltpu.make_async_copy(rep_tbl.at[src_sup.at[0]], rows[b],
                              sem_g[b]).wait()

    def wait_scatter(b):
        pltpu.make_async_copy(rows[b], acc_sh.at[dst_sup.at[0]],
                              sem_s[b]).wait()

    def super_body(sb, carry):
        row0 = base_row + sb * SUP
        pltpu.sync_copy(src_hbm.at[pl.ds(row0, SUP)], src_sup)
        pltpu.sync_copy(dst_hbm.at[pl.ds(row0, SUP)], dst_sup)
        pltpu.sync_copy(ew_hbm.at[pl.ds(row0, SUP)], ew_sup)
        start_gather(0, 0)

        def body(jj, carry2):
            for b in range(2):
                j = 2 * jj + b

                @pl.when(j + 1 < SUP)
                def _():
                    @pl.when(j >= 1)
                    def _():
                        wait_scatter(1 - b)

                    start_gather(j + 1, 1 - b)

                wait_gather(b)
                for g in range(CB // L):
                    sg = src_sup[j, pl.ds(g * L, L)]
                    dg = dst_sup[j, pl.ds(g * L, L)]
                    ewg = ew_sup[j, pl.ds(g * L, L)]
                    ng = (plsc.load_gather(dis_v, [sg]) * ewg
                          * plsc.load_gather(dis_v, [dg]))
                    nrm_v[pl.ds(g * L, L)] = ng

                rb = rows[b]

                def scale_row(i, carry3):
                    nspl = plsc.load_gather(
                        nrm_v, [jnp.zeros((L,), jnp.int32) + i])
                    for jv in range(d // L):
                        sl = pl.ds(jv * L, L)
                        rb[i, sl] = rb[i, sl] * nspl
                    return carry3

                lax.fori_loop(0, CB, scale_row, 0)
                pltpu.async_copy(rows[b], acc_sh.at[dst_sup.at[j]],
                                 sem_s[b], add=True)
            return carry2

        lax.fori_loop(0, SUP // 2, body, 0)
        wait_scatter(0)
        wait_scatter(1)
        return carry

    lax.fori_loop(0, rpt // SUP, super_body, 0)
    plsc.subcore_barrier()


def _make_agg_feat_split(n_nodes, n_pad, n_edges_pad, d_half):
    rpt = n_edges_pad // CB // NS
    r_tile = n_pad // NS
    assert r_tile % CB == 0 and d_half % L == 0 and rpt % SUP == 0

    @functools.partial(
        pl.kernel,
        out_type=[
            jax.ShapeDtypeStruct((n_pad, d_half), jnp.float32),
            jax.ShapeDtypeStruct((n_pad, d_half), jnp.float32),
        ],
        scratch_types=[
            pltpu.VMEM((SUP, CB), jnp.int32),
            pltpu.VMEM((SUP, CB), jnp.int32),
            pltpu.VMEM((SUP, CB), jnp.float32),
            pltpu.VMEM((CB,), jnp.float32),
            pltpu.VMEM((CB, d_half), jnp.float32),
            pltpu.VMEM((CB, d_half), jnp.float32),
            pltpu.VMEM((n_nodes,), jnp.float32),
            pltpu.VMEM_SHARED((n_pad, d_half), jnp.float32),
            pltpu.SemaphoreType.DMA,
            pltpu.SemaphoreType.DMA,
            pltpu.SemaphoreType.DMA,
            pltpu.SemaphoreType.DMA,
        ],
        **_SC_PARAMS,
    )
    def agg_kernel(xa_hbm, xb_hbm, src_hbm, dst_hbm, ew_hbm, dis_hbm,
                   oa_hbm, ob_hbm,
                   src_sup, dst_sup, ew_sup, nrm_v, rows0, rows1, dis_v,
                   acc_sh, sg0, sg1, ss0, ss1):
        c = lax.axis_index("c")
        s = lax.axis_index("s")
        rows = (rows0, rows1)
        sem_g = (sg0, sg1)
        sem_s = (ss0, ss1)

        def start_gather(j, b):
            @pl.when(c == 0)
            def _():
                pltpu.async_copy(xa_hbm.at[src_sup.at[j]], rows[b],
                                 sem_g[b])

            @pl.when(c == 1)
            def _():
                pltpu.async_copy(xb_hbm.at[src_sup.at[j]], rows[b],
                                 sem_g[b])

        _agg_pipeline(s, rpt, d_half, r_tile,
                      src_sup, dst_sup, ew_sup, nrm_v, rows, dis_v, acc_sh,
                      sem_g, sem_s,
                      dis_hbm, src_hbm, dst_hbm, ew_hbm, xa_hbm,
                      s * rpt, start_gather)

        ro = s * r_tile

        @pl.when(c == 0)
        def _():
            pltpu.sync_copy(acc_sh.at[pl.ds(ro, r_tile)],
                            oa_hbm.at[pl.ds(ro, r_tile)])

        @pl.when(c == 1)
        def _():
            pltpu.sync_copy(acc_sh.at[pl.ds(ro, r_tile)],
                            ob_hbm.at[pl.ds(ro, r_tile)])

    return agg_kernel


def _make_agg_edge_split(n_nodes, n_pad, n_edges_pad, d):
    rpt = n_edges_pad // CB // (NC * NS)
    r_tile = n_pad // NS
    assert r_tile % CB == 0 and d % L == 0 and rpt % SUP == 0

    @functools.partial(
        pl.kernel,
        mesh=_sc_mesh(),
        compiler_params=pltpu.CompilerParams(
            needs_layout_passes=False, use_tc_tiling_on_sc=False),
        out_type=jax.ShapeDtypeStruct((NC, n_pad, d), jnp.float32),
        scratch_types=[
            pltpu.VMEM((SUP, CB), jnp.int32),
            pltpu.VMEM((SUP, CB), jnp.int32),
            pltpu.VMEM((SUP, CB), jnp.float32),
            pltpu.VMEM((CB,), jnp.float32),
            pltpu.VMEM((CB, d), jnp.float32),
            pltpu.VMEM((CB, d), jnp.float32),
            pltpu.VMEM((n_nodes,), jnp.float32),
            pltpu.VMEM_SHARED((n_pad, d), jnp.float32),
            pltpu.SemaphoreType.DMA,
            pltpu.SemaphoreType.DMA,
            pltpu.SemaphoreType.DMA,
            pltpu.SemaphoreType.DMA,
        ],
    )
    def agg_kernel(xw_hbm, src_hbm, dst_hbm, ew_hbm, dis_hbm, out_hbm,
                   src_sup, dst_sup, ew_sup, nrm_v, rows0, rows1, dis_v,
                   acc_sh, sg0, sg1, ss0, ss1):
        c = lax.axis_index("c")
        s = lax.axis_index("s")
        rows = (rows0, rows1)
        sem_g = (sg0, sg1)
        sem_s = (ss0, ss1)

        def start_gather(j, b):
            pltpu.async_copy(xw_hbm.at[src_sup.at[j]], rows[b], sem_g[b])

        _agg_pipeline(s, rpt, d, r_tile,
                      src_sup, dst_sup, ew_sup, nrm_v, rows, dis_v, acc_sh,
                      sem_g, sem_s,
                      dis_hbm, src_hbm, dst_hbm, ew_hbm, xw_hbm,
                      (c * NS + s) * rpt, start_gather)

        ro = s * r_tile
        pltpu.sync_copy(acc_sh.at[pl.ds(ro, r_tile)],
                        out_hbm.at[c].at[pl.ds(ro, r_tile)])

    return agg_kernel


def _tc_degnorm(deg32):
    w, n_pad = deg32.shape

    def body(deg_ref, dis_ref, dinv_ref):
        d = 1.0 + jnp.sum(deg_ref[...], axis=0, keepdims=True)
        d = jnp.maximum(d, 1e-30)
        dis_ref[...] = lax.rsqrt(d)
        dinv_ref[...] = 1.0 / d

    return pl.pallas_call(
        body,
        out_shape=[
            jax.ShapeDtypeStruct((1, n_pad), jnp.float32),
            jax.ShapeDtypeStruct((1, n_pad), jnp.float32),
        ],
    )(deg32)


def _tc1(x, w1t, rows_blk):
    n, in_ch = x.shape
    hid = w1t.shape[1]
    h2 = hid // 2
    grid = n // rows_blk

    def body(x_ref, w_ref, xa_ref, xb_ref):
        xw = jnp.dot(x_ref[...], w_ref[...],
                     preferred_element_type=jnp.float32)
        xa_ref[...] = xw[:, :h2]
        xb_ref[...] = xw[:, h2:]

    return pl.pallas_call(
        body,
        grid=(grid,),
        in_specs=[
            pl.BlockSpec((rows_blk, in_ch), lambda i: (i, 0)),
            pl.BlockSpec((in_ch, hid), lambda i: (0, 0)),
        ],
        out_specs=[
            pl.BlockSpec((rows_blk, h2), lambda i: (i, 0)),
            pl.BlockSpec((rows_blk, h2), lambda i: (i, 0)),
        ],
        out_shape=[
            jax.ShapeDtypeStruct((n, h2), jnp.float32),
            jax.ShapeDtypeStruct((n, h2), jnp.float32),
        ],
    )(x, w1t)


def _tc2(aa, ab, xa, xb, dinv, b1, w2t, d_pad, rows_blk):
    n, h2 = xa.shape
    hid = 2 * h2
    out_ch = w2t.shape[1]
    grid = n // rows_blk

    def body(aa_ref, ab_ref, xa_ref, xb_ref, dinv_ref, b1_ref, w_ref,
             hw_ref):
        dv = dinv_ref[...]
        ha = jnp.maximum(aa_ref[...] + dv * xa_ref[...] + b1_ref[0, :h2], 0.0)
        hb = jnp.maximum(ab_ref[...] + dv * xb_ref[...] + b1_ref[0, h2:], 0.0)
        h = jnp.concatenate([ha, hb], axis=1)
        hw = jnp.dot(h, w_ref[...], preferred_element_type=jnp.float32)
        if d_pad > out_ch:
            hw = jnp.concatenate(
                [hw, jnp.zeros((rows_blk, d_pad - out_ch), jnp.float32)],
                axis=1)
        hw_ref[...] = hw

    return pl.pallas_call(
        body,
        grid=(grid,),
        in_specs=[
            pl.BlockSpec((rows_blk, h2), lambda i: (i, 0)),
            pl.BlockSpec((rows_blk, h2), lambda i: (i, 0)),
            pl.BlockSpec((rows_blk, h2), lambda i: (i, 0)),
            pl.BlockSpec((rows_blk, h2), lambda i: (i, 0)),
            pl.BlockSpec((rows_blk, 1), lambda i: (i, 0)),
            pl.BlockSpec((1, hid), lambda i: (0, 0)),
            pl.BlockSpec((hid, out_ch), lambda i: (0, 0)),
        ],
        out_specs=pl.BlockSpec((rows_blk, d_pad), lambda i: (i, 0)),
        out_shape=jax.ShapeDtypeStruct((n, d_pad), jnp.float32),
    )(aa, ab, xa, xb, dinv, b1, w2t)


def _tc3(g2, hw, dinv, b2, out_ch, rows_blk):
    n, d_pad = hw.shape
    grid = n // rows_blk

    def body(g_ref, hw_ref, dinv_ref, b2_ref, out_ref):
        gsum = jnp.sum(g_ref[...], axis=0)
        zf = gsum + dinv_ref[...] * hw_ref[...]
        z = zf[:, :out_ch] + b2_ref[0, :]
        m = jnp.max(z, axis=1, keepdims=True)
        e = z - m
        lse = jnp.log(jnp.sum(jnp.exp(e), axis=1, keepdims=True))
        out_ref[...] = e - lse

    return pl.pallas_call(
        body,
        grid=(grid,),
        in_specs=[
            pl.BlockSpec((NC, rows_blk, d_pad), lambda i: (0, i, 0)),
            pl.BlockSpec((rows_blk, d_pad), lambda i: (i, 0)),
            pl.BlockSpec((rows_blk, 1), lambda i: (i, 0)),
            pl.BlockSpec((1, out_ch), lambda i: (0, 0)),
        ],
        out_specs=pl.BlockSpec((rows_blk, out_ch), lambda i: (i, 0)),
        out_shape=jax.ShapeDtypeStruct((n, out_ch), jnp.float32),
    )(g2, hw, dinv, b2)


@jax.jit
def kernel(x, edge_index, edge_weight, W1, b1, W2, b2):
    n, _ = x.shape
    hid = W1.shape[0]
    out_ch = W2.shape[0]
    e = edge_index.shape[1]
    d_pad = out_ch

    src = edge_index[0].astype(jnp.int32)
    dst = edge_index[1].astype(jnp.int32)
    ew = edge_weight.astype(jnp.float32)

    ep = ((e + NC * NS * CB - 1) // (NC * NS * CB)) * (NC * NS * CB)
    pad = ep - e
    if pad:
        src = jnp.concatenate([src, jnp.zeros((pad,), jnp.int32)])
        dst = jnp.concatenate([dst, jnp.zeros((pad,), jnp.int32)])
        ew = jnp.concatenate([ew, jnp.zeros((pad,), jnp.float32)])
    src2 = src.reshape(ep // CB, CB)
    dst2 = dst.reshape(ep // CB, CB)
    ew2 = ew.reshape(ep // CB, CB)

    rows_blk = 1000
    n_pad = ((n + NS * CB - 1) // (NS * CB)) * (NS * CB)

    deg32 = _make_deg_kernel(n_pad, ep)(dst2, ew2)
    dis_r, dinv_r = _tc_degnorm(deg32.reshape(NC * NS, n_pad))
    dis = dis_r.reshape(n_pad)[:n]
    dinv = dinv_r.reshape(n_pad)[:n].reshape(n, 1)

    xa, xb = _tc1(x, W1.T, rows_blk)
    aa, ab = _make_agg_feat_split(n, n_pad, ep, hid // 2)(
        xa, xb, src2, dst2, ew2, dis)
    hw = _tc2(aa, ab, xa, xb, dinv, b1.reshape(1, hid), W2.T, d_pad, rows_blk)
    g2 = _make_agg_edge_split(n, n_pad, ep, d_pad)(
        hw, src2, dst2, ew2, dis)
    return _tc3(g2, hw, dinv, b2.reshape(1, out_ch), out_ch, rows_blk)


# speedup vs baseline: 8.7929x; 1.0572x over previous
import functools

import jax
import jax.numpy as jnp
from jax import lax
from jax.experimental import pallas as pl
from jax.experimental.pallas import tpu as pltpu
from jax.experimental.pallas import tpu_sc as plsc

NC = 2
NS = 16
L = 16
CB = 128


def _sc_mesh():
    return plsc.VectorSubcoreMesh(core_axis_name="c", subcore_axis_name="s")


_SC_PARAMS = dict(
    mesh=_sc_mesh(),
    compiler_params=pltpu.CompilerParams(needs_layout_passes=False),
)


def _zero_rows(ref, n_rows, n_cols):
    zeros = jnp.zeros((L,), jnp.float32)

    def body(i, carry):
        for j in range(n_cols // L):
            ref[i, pl.ds(j * L, L)] = zeros
        return carry

    lax.fori_loop(0, n_rows, body, 0)


def _make_deg_kernel(n_pad, n_edges_pad):
    rpt = n_edges_pad // CB // (NC * NS)

    @functools.partial(
        pl.kernel,
        out_type=jax.ShapeDtypeStruct((NC, NS, n_pad), jnp.float32),
        scratch_types=[
            pltpu.VMEM((rpt, CB), jnp.int32),
            pltpu.VMEM((rpt, CB), jnp.float32),
            pltpu.VMEM((n_pad,), jnp.float32),
        ],
        **_SC_PARAMS,
    )
    def deg_kernel(dst_hbm, ew_hbm, out_hbm, dst_sup, ew_sup, deg_v):
        c = lax.axis_index("c")
        s = lax.axis_index("s")
        base_row = (c * NS + s) * rpt
        pltpu.sync_copy(dst_hbm.at[pl.ds(base_row, rpt)], dst_sup)
        pltpu.sync_copy(ew_hbm.at[pl.ds(base_row, rpt)], ew_sup)
        zeros = jnp.zeros((L,), jnp.float32)

        def zbody(i, carry):
            deg_v[pl.ds(i * L, L)] = zeros
            return carry

        lax.fori_loop(0, n_pad // L, zbody, 0)
        lane = lax.iota(jnp.int32, L)

        def chunk(k, carry):
            for g in range(CB // L):
                dg = dst_sup[k, pl.ds(g * L, L)]
                ewg = ew_sup[k, pl.ds(g * L, L)]
                for l in range(L):
                    plsc.addupdate_scatter(deg_v, [dg], ewg,
                                           mask=lane == l)
            return carry

        lax.fori_loop(0, rpt, chunk, 0)
        pltpu.sync_copy(deg_v, out_hbm.at[c].at[s])

    return deg_kernel


SUP = 8


def _agg_pipeline(s, rpt, d, r_tile,
                  src_sup, dst_sup, ew_sup, nrm_v, rows, dis_v, acc_sh,
                  sem_g, sem_s,
                  dis_hbm, src_hbm, dst_hbm, ew_hbm, rep_tbl,
                  base_row, start_gather):
    pltpu.sync_copy(dis_hbm, dis_v)

    _zero_rows(rows[1], CB, d)
    for k in range(r_tile // CB):
        pltpu.sync_copy(rows[1], acc_sh.at[pl.ds(s * r_tile + k * CB, CB)])
    plsc.subcore_barrier()

    def wait_gather(b):
        pltpu.make_async_copy(rep_tbl.at[src_sup.at[0]], rows[b],
                              sem_g[b]).wait()

    def wait_scatter(b):
        pltpu.make_async_copy(rows[b], acc_sh.at[dst_sup.at[0]],
                              sem_s[b]).wait()

    def super_body(sb, carry):
        row0 = base_row + sb * SUP
        pltpu.sync_copy(src_hbm.at[pl.ds(row0, SUP)], src_sup)
        pltpu.sync_copy(dst_hbm.at[pl.ds(row0, SUP)], dst_sup)
        pltpu.sync_copy(ew_hbm.at[pl.ds(row0, SUP)], ew_sup)
        start_gather(0, 0)

        def body(jj, carry2):
            for b in range(2):
                j = 2 * jj + b

                @pl.when(j + 1 < SUP)
                def _():
                    @pl.when(j >= 1)
                    def _():
                        wait_scatter(1 - b)

                    start_gather(j + 1, 1 - b)

                wait_gather(b)
                for g in range(CB // L):
                    sg = src_sup[j, pl.ds(g * L, L)]
                    dg = dst_sup[j, pl.ds(g * L, L)]
                    ewg = ew_sup[j, pl.ds(g * L, L)]
                    ng = (plsc.load_gather(dis_v, [sg]) * ewg
                          * plsc.load_gather(dis_v, [dg]))
                    nrm_v[pl.ds(g * L, L)] = ng

                rb = rows[b]

                @plsc.parallel_loop(0, CB, 1, unroll=4)
                def _(i):
                    nspl = plsc.load_gather(
                        nrm_v, [jnp.zeros((L,), jnp.int32) + i])
                    for jv in range(d // L):
                        sl = pl.ds(jv * L, L)
                        rb[i, sl] = rb[i, sl] * nspl
                pltpu.async_copy(rows[b], acc_sh.at[dst_sup.at[j]],
                                 sem_s[b], add=True)
            return carry2

        lax.fori_loop(0, SUP // 2, body, 0)
        wait_scatter(0)
        wait_scatter(1)
        return carry

    lax.fori_loop(0, rpt // SUP, super_body, 0)
    plsc.subcore_barrier()


def _make_agg_feat_split(n_nodes, n_pad, n_edges_pad, d_half):
    rpt = n_edges_pad // CB // NS
    r_tile = n_pad // NS
    assert r_tile % CB == 0 and d_half % L == 0 and rpt % SUP == 0

    @functools.partial(
        pl.kernel,
        out_type=[
            jax.ShapeDtypeStruct((n_pad, d_half), jnp.float32),
            jax.ShapeDtypeStruct((n_pad, d_half), jnp.float32),
        ],
        scratch_types=[
            pltpu.VMEM((SUP, CB), jnp.int32),
            pltpu.VMEM((SUP, CB), jnp.int32),
            pltpu.VMEM((SUP, CB), jnp.float32),
            pltpu.VMEM((CB,), jnp.float32),
            pltpu.VMEM((CB, d_half), jnp.float32),
            pltpu.VMEM((CB, d_half), jnp.float32),
            pltpu.VMEM((n_nodes,), jnp.float32),
            pltpu.VMEM_SHARED((n_pad, d_half), jnp.float32),
            pltpu.SemaphoreType.DMA,
            pltpu.SemaphoreType.DMA,
            pltpu.SemaphoreType.DMA,
            pltpu.SemaphoreType.DMA,
        ],
        **_SC_PARAMS,
    )
    def agg_kernel(xa_hbm, xb_hbm, src_hbm, dst_hbm, ew_hbm, dis_hbm,
                   oa_hbm, ob_hbm,
                   src_sup, dst_sup, ew_sup, nrm_v, rows0, rows1, dis_v,
                   acc_sh, sg0, sg1, ss0, ss1):
        c = lax.axis_index("c")
        s = lax.axis_index("s")
        rows = (rows0, rows1)
        sem_g = (sg0, sg1)
        sem_s = (ss0, ss1)

        def start_gather(j, b):
            @pl.when(c == 0)
            def _():
                pltpu.async_copy(xa_hbm.at[src_sup.at[j]], rows[b],
                                 sem_g[b])

            @pl.when(c == 1)
            def _():
                pltpu.async_copy(xb_hbm.at[src_sup.at[j]], rows[b],
                                 sem_g[b])

        _agg_pipeline(s, rpt, d_half, r_tile,
                      src_sup, dst_sup, ew_sup, nrm_v, rows, dis_v, acc_sh,
                      sem_g, sem_s,
                      dis_hbm, src_hbm, dst_hbm, ew_hbm, xa_hbm,
                      s * rpt, start_gather)

        ro = s * r_tile

        @pl.when(c == 0)
        def _():
            pltpu.sync_copy(acc_sh.at[pl.ds(ro, r_tile)],
                            oa_hbm.at[pl.ds(ro, r_tile)])

        @pl.when(c == 1)
        def _():
            pltpu.sync_copy(acc_sh.at[pl.ds(ro, r_tile)],
                            ob_hbm.at[pl.ds(ro, r_tile)])

    return agg_kernel


def _make_agg_edge_split(n_nodes, n_pad, n_edges_pad, d):
    rpt = n_edges_pad // CB // (NC * NS)
    r_tile = n_pad // NS
    assert r_tile % CB == 0 and d % L == 0 and rpt % SUP == 0

    @functools.partial(
        pl.kernel,
        mesh=_sc_mesh(),
        compiler_params=pltpu.CompilerParams(
            needs_layout_passes=False, use_tc_tiling_on_sc=False),
        out_type=jax.ShapeDtypeStruct((NC, n_pad, d), jnp.float32),
        scratch_types=[
            pltpu.VMEM((SUP, CB), jnp.int32),
            pltpu.VMEM((SUP, CB), jnp.int32),
            pltpu.VMEM((SUP, CB), jnp.float32),
            pltpu.VMEM((CB,), jnp.float32),
            pltpu.VMEM((CB, d), jnp.float32),
            pltpu.VMEM((CB, d), jnp.float32),
            pltpu.VMEM((n_nodes,), jnp.float32),
            pltpu.VMEM_SHARED((n_pad, d), jnp.float32),
            pltpu.SemaphoreType.DMA,
            pltpu.SemaphoreType.DMA,
            pltpu.SemaphoreType.DMA,
            pltpu.SemaphoreType.DMA,
        ],
    )
    def agg_kernel(xw_hbm, src_hbm, dst_hbm, ew_hbm, dis_hbm, out_hbm,
                   src_sup, dst_sup, ew_sup, nrm_v, rows0, rows1, dis_v,
                   acc_sh, sg0, sg1, ss0, ss1):
        c = lax.axis_index("c")
        s = lax.axis_index("s")
        rows = (rows0, rows1)
        sem_g = (sg0, sg1)
        sem_s = (ss0, ss1)

        def start_gather(j, b):
            pltpu.async_copy(xw_hbm.at[src_sup.at[j]], rows[b], sem_g[b])

        _agg_pipeline(s, rpt, d, r_tile,
                      src_sup, dst_sup, ew_sup, nrm_v, rows, dis_v, acc_sh,
                      sem_g, sem_s,
                      dis_hbm, src_hbm, dst_hbm, ew_hbm, xw_hbm,
                      (c * NS + s) * rpt, start_gather)

        ro = s * r_tile
        pltpu.sync_copy(acc_sh.at[pl.ds(ro, r_tile)],
                        out_hbm.at[c].at[pl.ds(ro, r_tile)])

    return agg_kernel


def _tc_degnorm(deg32):
    w, n_pad = deg32.shape

    def body(deg_ref, dis_ref, dinv_ref):
        d = 1.0 + jnp.sum(deg_ref[...], axis=0, keepdims=True)
        d = jnp.maximum(d, 1e-30)
        dis_ref[...] = lax.rsqrt(d)
        dinv_ref[...] = 1.0 / d

    return pl.pallas_call(
        body,
        out_shape=[
            jax.ShapeDtypeStruct((1, n_pad), jnp.float32),
            jax.ShapeDtypeStruct((1, n_pad), jnp.float32),
        ],
    )(deg32)


def _tc1(x, w1t, rows_blk):
    n, in_ch = x.shape
    hid = w1t.shape[1]
    h2 = hid // 2
    grid = n // rows_blk

    def body(x_ref, w_ref, xa_ref, xb_ref):
        xw = jnp.dot(x_ref[...], w_ref[...],
                     preferred_element_type=jnp.float32)
        xa_ref[...] = xw[:, :h2]
        xb_ref[...] = xw[:, h2:]

    return pl.pallas_call(
        body,
        grid=(grid,),
        in_specs=[
            pl.BlockSpec((rows_blk, in_ch), lambda i: (i, 0)),
            pl.BlockSpec((in_ch, hid), lambda i: (0, 0)),
        ],
        out_specs=[
            pl.BlockSpec((rows_blk, h2), lambda i: (i, 0)),
            pl.BlockSpec((rows_blk, h2), lambda i: (i, 0)),
        ],
        out_shape=[
            jax.ShapeDtypeStruct((n, h2), jnp.float32),
            jax.ShapeDtypeStruct((n, h2), jnp.float32),
        ],
    )(x, w1t)


def _tc2(aa, ab, xa, xb, dinv, b1, w2t, d_pad, rows_blk):
    n, h2 = xa.shape
    hid = 2 * h2
    out_ch = w2t.shape[1]
    grid = n // rows_blk

    def body(aa_ref, ab_ref, xa_ref, xb_ref, dinv_ref, b1_ref, w_ref,
             hw_ref):
        dv = dinv_ref[...]
        ha = jnp.maximum(aa_ref[...] + dv * xa_ref[...] + b1_ref[0, :h2], 0.0)
        hb = jnp.maximum(ab_ref[...] + dv * xb_ref[...] + b1_ref[0, h2:], 0.0)
        h = jnp.concatenate([ha, hb], axis=1)
        hw = jnp.dot(h, w_ref[...], preferred_element_type=jnp.float32)
        if d_pad > out_ch:
            hw = jnp.concatenate(
                [hw, jnp.zeros((rows_blk, d_pad - out_ch), jnp.float32)],
                axis=1)
        hw_ref[...] = hw

    return pl.pallas_call(
        body,
        grid=(grid,),
        in_specs=[
            pl.BlockSpec((rows_blk, h2), lambda i: (i, 0)),
            pl.BlockSpec((rows_blk, h2), lambda i: (i, 0)),
            pl.BlockSpec((rows_blk, h2), lambda i: (i, 0)),
            pl.BlockSpec((rows_blk, h2), lambda i: (i, 0)),
            pl.BlockSpec((rows_blk, 1), lambda i: (i, 0)),
            pl.BlockSpec((1, hid), lambda i: (0, 0)),
            pl.BlockSpec((hid, out_ch), lambda i: (0, 0)),
        ],
        out_specs=pl.BlockSpec((rows_blk, d_pad), lambda i: (i, 0)),
        out_shape=jax.ShapeDtypeStruct((n, d_pad), jnp.float32),
    )(aa, ab, xa, xb, dinv, b1, w2t)


def _tc3(g2, hw, dinv, b2, out_ch, rows_blk):
    n, d_pad = hw.shape
    grid = n // rows_blk

    def body(g_ref, hw_ref, dinv_ref, b2_ref, out_ref):
        gsum = jnp.sum(g_ref[...], axis=0)
        zf = gsum + dinv_ref[...] * hw_ref[...]
        z = zf[:, :out_ch] + b2_ref[0, :]
        m = jnp.max(z, axis=1, keepdims=True)
        e = z - m
        lse = jnp.log(jnp.sum(jnp.exp(e), axis=1, keepdims=True))
        out_ref[...] = e - lse

    return pl.pallas_call(
        body,
        grid=(grid,),
        in_specs=[
            pl.BlockSpec((NC, rows_blk, d_pad), lambda i: (0, i, 0)),
            pl.BlockSpec((rows_blk, d_pad), lambda i: (i, 0)),
            pl.BlockSpec((rows_blk, 1), lambda i: (i, 0)),
            pl.BlockSpec((1, out_ch), lambda i: (0, 0)),
        ],
        out_specs=pl.BlockSpec((rows_blk, out_ch), lambda i: (i, 0)),
        out_shape=jax.ShapeDtypeStruct((n, out_ch), jnp.float32),
    )(g2, hw, dinv, b2)


@jax.jit
def kernel(x, edge_index, edge_weight, W1, b1, W2, b2):
    n, _ = x.shape
    hid = W1.shape[0]
    out_ch = W2.shape[0]
    e = edge_index.shape[1]
    d_pad = out_ch

    src = edge_index[0].astype(jnp.int32)
    dst = edge_index[1].astype(jnp.int32)
    ew = edge_weight.astype(jnp.float32)

    ep = ((e + NC * NS * CB - 1) // (NC * NS * CB)) * (NC * NS * CB)
    pad = ep - e
    if pad:
        src = jnp.concatenate([src, jnp.zeros((pad,), jnp.int32)])
        dst = jnp.concatenate([dst, jnp.zeros((pad,), jnp.int32)])
        ew = jnp.concatenate([ew, jnp.zeros((pad,), jnp.float32)])
    src2 = src.reshape(ep // CB, CB)
    dst2 = dst.reshape(ep // CB, CB)
    ew2 = ew.reshape(ep // CB, CB)

    rows_blk = 1000
    n_pad = ((n + NS * CB - 1) // (NS * CB)) * (NS * CB)

    deg32 = _make_deg_kernel(n_pad, ep)(dst2, ew2)
    dis_r, dinv_r = _tc_degnorm(deg32.reshape(NC * NS, n_pad))
    dis = dis_r.reshape(n_pad)[:n]
    dinv = dinv_r.reshape(n_pad)[:n].reshape(n, 1)

    xa, xb = _tc1(x, W1.T, rows_blk)
    aa, ab = _make_agg_feat_split(n, n_pad, ep, hid // 2)(
        xa, xb, src2, dst2, ew2, dis)
    hw = _tc2(aa, ab, xa, xb, dinv, b1.reshape(1, hid), W2.T, d_pad, rows_blk)
    g2 = _make_agg_edge_split(n, n_pad, ep, d_pad)(
        hw, src2, dst2, ew2, dis)
    return _tc3(g2, hw, dinv, b2.reshape(1, out_ch), out_ch, rows_blk)


# speedup vs baseline: 9.0055x; 1.0242x over previous
import functools

import jax
import jax.numpy as jnp
from jax import lax
from jax.experimental import pallas as pl
from jax.experimental.pallas import tpu as pltpu
from jax.experimental.pallas import tpu_sc as plsc

NC = 2
NS = 16
L = 16
CB = 128


def _sc_mesh():
    return plsc.VectorSubcoreMesh(core_axis_name="c", subcore_axis_name="s")


_SC_PARAMS = dict(
    mesh=_sc_mesh(),
    compiler_params=pltpu.CompilerParams(needs_layout_passes=False),
)


def _zero_rows(ref, n_rows, n_cols):
    zeros = jnp.zeros((L,), jnp.float32)

    def body(i, carry):
        for j in range(n_cols // L):
            ref[i, pl.ds(j * L, L)] = zeros
        return carry

    lax.fori_loop(0, n_rows, body, 0)


def _make_deg_kernel(n_pad, n_edges_pad):
    rpt = n_edges_pad // CB // (NC * NS)

    @functools.partial(
        pl.kernel,
        out_type=jax.ShapeDtypeStruct((NC, NS, n_pad), jnp.float32),
        scratch_types=[
            pltpu.VMEM((rpt, CB), jnp.int32),
            pltpu.VMEM((rpt, CB), jnp.float32),
            pltpu.VMEM((n_pad,), jnp.float32),
        ],
        **_SC_PARAMS,
    )
    def deg_kernel(dst_hbm, ew_hbm, out_hbm, dst_sup, ew_sup, deg_v):
        c = lax.axis_index("c")
        s = lax.axis_index("s")
        base_row = (c * NS + s) * rpt
        pltpu.sync_copy(dst_hbm.at[pl.ds(base_row, rpt)], dst_sup)
        pltpu.sync_copy(ew_hbm.at[pl.ds(base_row, rpt)], ew_sup)
        zeros = jnp.zeros((L,), jnp.float32)

        def zbody(i, carry):
            deg_v[pl.ds(i * L, L)] = zeros
            return carry

        lax.fori_loop(0, n_pad // L, zbody, 0)
        lane = lax.iota(jnp.int32, L)

        def chunk(k, carry):
            for g in range(CB // L):
                dg = dst_sup[k, pl.ds(g * L, L)]
                ewg = ew_sup[k, pl.ds(g * L, L)]
                for l in range(L):
                    plsc.addupdate_scatter(deg_v, [dg], ewg,
                                           mask=lane == l)
            return carry

        lax.fori_loop(0, rpt, chunk, 0)
        pltpu.sync_copy(deg_v, out_hbm.at[c].at[s])

    return deg_kernel


SUP = 8


def _agg_pipeline(s, rpt, d, r_tile, sup,
                  src_sup, dst_sup, ew_sup, nrm_v, rows, dis_v, acc_sh,
                  sem_g, sem_s,
                  dis_hbm, src_hbm, dst_hbm, ew_hbm, rep_tbl,
                  base_row, start_gather):
    pltpu.sync_copy(dis_hbm, dis_v)

    _zero_rows(rows[1], CB, d)
    for k in range(r_tile // CB):
        pltpu.sync_copy(rows[1], acc_sh.at[pl.ds(s * r_tile + k * CB, CB)])
    plsc.subcore_barrier()

    def wait_gather(b):
        pltpu.make_async_copy(rep_tbl.at[src_sup.at[0]], rows[b],
                              sem_g[b]).wait()

    def wait_scatter(b):
        pltpu.make_async_copy(rows[b], acc_sh.at[dst_sup.at[0]],
                              sem_s[b]).wait()

    def super_body(sb, carry):
        row0 = base_row + sb * sup
        pltpu.sync_copy(src_hbm.at[pl.ds(row0, sup)], src_sup)
        pltpu.sync_copy(dst_hbm.at[pl.ds(row0, sup)], dst_sup)
        pltpu.sync_copy(ew_hbm.at[pl.ds(row0, sup)], ew_sup)
        start_gather(0, 0)

        def body(jj, carry2):
            for b in range(2):
                j = 2 * jj + b

                @pl.when(j + 1 < sup)
                def _():
                    @pl.when(j >= 1)
                    def _():
                        wait_scatter(1 - b)

                    start_gather(j + 1, 1 - b)

                wait_gather(b)
                for g in range(CB // L):
                    sg = src_sup[j, pl.ds(g * L, L)]
                    dg = dst_sup[j, pl.ds(g * L, L)]
                    ewg = ew_sup[j, pl.ds(g * L, L)]
                    ng = (plsc.load_gather(dis_v, [sg]) * ewg
                          * plsc.load_gather(dis_v, [dg]))
                    nrm_v[pl.ds(g * L, L)] = ng

                rb = rows[b]

                @plsc.parallel_loop(0, CB, 1, unroll=4)
                def _(i):
                    nspl = plsc.load_gather(
                        nrm_v, [jnp.zeros((L,), jnp.int32) + i])
                    for jv in range(d // L):
                        sl = pl.ds(jv * L, L)
                        rb[i, sl] = rb[i, sl] * nspl
                pltpu.async_copy(rows[b], acc_sh.at[dst_sup.at[j]],
                                 sem_s[b], add=True)
            return carry2

        lax.fori_loop(0, sup // 2, body, 0)
        wait_scatter(0)
        wait_scatter(1)
        return carry

    lax.fori_loop(0, rpt // sup, super_body, 0)
    plsc.subcore_barrier()


def _make_agg_feat_split(n_nodes, n_pad, n_edges_pad, d_half):
    rpt = n_edges_pad // CB // NS
    r_tile = n_pad // NS
    assert r_tile % CB == 0 and d_half % L == 0 and rpt % SUP == 0

    @functools.partial(
        pl.kernel,
        out_type=[
            jax.ShapeDtypeStruct((n_pad, d_half), jnp.float32),
            jax.ShapeDtypeStruct((n_pad, d_half), jnp.float32),
        ],
        scratch_types=[
            pltpu.VMEM((SUP, CB), jnp.int32),
            pltpu.VMEM((SUP, CB), jnp.int32),
            pltpu.VMEM((SUP, CB), jnp.float32),
            pltpu.VMEM((CB,), jnp.float32),
            pltpu.VMEM((CB, d_half), jnp.float32),
            pltpu.VMEM((CB, d_half), jnp.float32),
            pltpu.VMEM((n_nodes,), jnp.float32),
            pltpu.VMEM_SHARED((n_pad, d_half), jnp.float32),
            pltpu.SemaphoreType.DMA,
            pltpu.SemaphoreType.DMA,
            pltpu.SemaphoreType.DMA,
            pltpu.SemaphoreType.DMA,
        ],
        **_SC_PARAMS,
    )
    def agg_kernel(xa_hbm, xb_hbm, src_hbm, dst_hbm, ew_hbm, dis_hbm,
                   oa_hbm, ob_hbm,
                   src_sup, dst_sup, ew_sup, nrm_v, rows0, rows1, dis_v,
                   acc_sh, sg0, sg1, ss0, ss1):
        c = lax.axis_index("c")
        s = lax.axis_index("s")
        rows = (rows0, rows1)
        sem_g = (sg0, sg1)
        sem_s = (ss0, ss1)

        def start_gather(j, b):
            @pl.when(c == 0)
            def _():
                pltpu.async_copy(xa_hbm.at[src_sup.at[j]], rows[b],
                                 sem_g[b])

            @pl.when(c == 1)
            def _():
                pltpu.async_copy(xb_hbm.at[src_sup.at[j]], rows[b],
                                 sem_g[b])

        _agg_pipeline(s, rpt, d_half, r_tile, SUP,
                      src_sup, dst_sup, ew_sup, nrm_v, rows, dis_v, acc_sh,
                      sem_g, sem_s,
                      dis_hbm, src_hbm, dst_hbm, ew_hbm, xa_hbm,
                      s * rpt, start_gather)

        ro = s * r_tile

        @pl.when(c == 0)
        def _():
            pltpu.sync_copy(acc_sh.at[pl.ds(ro, r_tile)],
                            oa_hbm.at[pl.ds(ro, r_tile)])

        @pl.when(c == 1)
        def _():
            pltpu.sync_copy(acc_sh.at[pl.ds(ro, r_tile)],
                            ob_hbm.at[pl.ds(ro, r_tile)])

    return agg_kernel


def _make_agg_edge_split(n_nodes, n_pad, n_edges_pad, d):
    rpt = n_edges_pad // CB // (NC * NS)
    r_tile = n_pad // NS
    assert r_tile % CB == 0 and d % L == 0 and rpt % SUP == 0

    @functools.partial(
        pl.kernel,
        mesh=_sc_mesh(),
        compiler_params=pltpu.CompilerParams(
            needs_layout_passes=False, use_tc_tiling_on_sc=False),
        out_type=jax.ShapeDtypeStruct((NC, n_pad, d), jnp.float32),
        scratch_types=[
            pltpu.VMEM((rpt, CB), jnp.int32),
            pltpu.VMEM((rpt, CB), jnp.int32),
            pltpu.VMEM((rpt, CB), jnp.float32),
            pltpu.VMEM((CB,), jnp.float32),
            pltpu.VMEM((CB, d), jnp.float32),
            pltpu.VMEM((CB, d), jnp.float32),
            pltpu.VMEM((n_nodes,), jnp.float32),
            pltpu.VMEM_SHARED((n_pad, d), jnp.float32),
            pltpu.SemaphoreType.DMA,
            pltpu.SemaphoreType.DMA,
            pltpu.SemaphoreType.DMA,
            pltpu.SemaphoreType.DMA,
        ],
    )
    def agg_kernel(xw_hbm, src_hbm, dst_hbm, ew_hbm, dis_hbm, out_hbm,
                   src_sup, dst_sup, ew_sup, nrm_v, rows0, rows1, dis_v,
                   acc_sh, sg0, sg1, ss0, ss1):
        c = lax.axis_index("c")
        s = lax.axis_index("s")
        rows = (rows0, rows1)
        sem_g = (sg0, sg1)
        sem_s = (ss0, ss1)

        def start_gather(j, b):
            pltpu.async_copy(xw_hbm.at[src_sup.at[j]], rows[b], sem_g[b])

        _agg_pipeline(s, rpt, d, r_tile, rpt,
                      src_sup, dst_sup, ew_sup, nrm_v, rows, dis_v, acc_sh,
                      sem_g, sem_s,
                      dis_hbm, src_hbm, dst_hbm, ew_hbm, xw_hbm,
                      (c * NS + s) * rpt, start_gather)

        ro = s * r_tile
        pltpu.sync_copy(acc_sh.at[pl.ds(ro, r_tile)],
                        out_hbm.at[c].at[pl.ds(ro, r_tile)])

    return agg_kernel


def _tc_degnorm(deg32):
    w, n_pad = deg32.shape

    def body(deg_ref, dis_ref, dinv_ref):
        d = 1.0 + jnp.sum(deg_ref[...], axis=0, keepdims=True)
        d = jnp.maximum(d, 1e-30)
        dis_ref[...] = lax.rsqrt(d)
        dinv_ref[...] = 1.0 / d

    return pl.pallas_call(
        body,
        out_shape=[
            jax.ShapeDtypeStruct((1, n_pad), jnp.float32),
            jax.ShapeDtypeStruct((1, n_pad), jnp.float32),
        ],
    )(deg32)


def _tc1(x, w1t, rows_blk):
    n, in_ch = x.shape
    hid = w1t.shape[1]
    h2 = hid // 2
    grid = n // rows_blk

    def body(x_ref, w_ref, xa_ref, xb_ref):
        xw = jnp.dot(x_ref[...], w_ref[...],
                     preferred_element_type=jnp.float32)
        xa_ref[...] = xw[:, :h2]
        xb_ref[...] = xw[:, h2:]

    return pl.pallas_call(
        body,
        grid=(grid,),
        in_specs=[
            pl.BlockSpec((rows_blk, in_ch), lambda i: (i, 0)),
            pl.BlockSpec((in_ch, hid), lambda i: (0, 0)),
        ],
        out_specs=[
            pl.BlockSpec((rows_blk, h2), lambda i: (i, 0)),
            pl.BlockSpec((rows_blk, h2), lambda i: (i, 0)),
        ],
        out_shape=[
            jax.ShapeDtypeStruct((n, h2), jnp.float32),
            jax.ShapeDtypeStruct((n, h2), jnp.float32),
        ],
    )(x, w1t)


def _tc2(aa, ab, xa, xb, dinv, b1, w2t, d_pad, rows_blk):
    n, h2 = xa.shape
    hid = 2 * h2
    out_ch = w2t.shape[1]
    grid = n // rows_blk

    def body(aa_ref, ab_ref, xa_ref, xb_ref, dinv_ref, b1_ref, w_ref,
             hw_ref):
        dv = dinv_ref[...]
        ha = jnp.maximum(aa_ref[...] + dv * xa_ref[...] + b1_ref[0, :h2], 0.0)
        hb = jnp.maximum(ab_ref[...] + dv * xb_ref[...] + b1_ref[0, h2:], 0.0)
        h = jnp.concatenate([ha, hb], axis=1)
        hw = jnp.dot(h, w_ref[...], preferred_element_type=jnp.float32)
        if d_pad > out_ch:
            hw = jnp.concatenate(
                [hw, jnp.zeros((rows_blk, d_pad - out_ch), jnp.float32)],
                axis=1)
        hw_ref[...] = hw

    return pl.pallas_call(
        body,
        grid=(grid,),
        in_specs=[
            pl.BlockSpec((rows_blk, h2), lambda i: (i, 0)),
            pl.BlockSpec((rows_blk, h2), lambda i: (i, 0)),
            pl.BlockSpec((rows_blk, h2), lambda i: (i, 0)),
            pl.BlockSpec((rows_blk, h2), lambda i: (i, 0)),
            pl.BlockSpec((rows_blk, 1), lambda i: (i, 0)),
            pl.BlockSpec((1, hid), lambda i: (0, 0)),
            pl.BlockSpec((hid, out_ch), lambda i: (0, 0)),
        ],
        out_specs=pl.BlockSpec((rows_blk, d_pad), lambda i: (i, 0)),
        out_shape=jax.ShapeDtypeStruct((n, d_pad), jnp.float32),
    )(aa, ab, xa, xb, dinv, b1, w2t)


def _tc3(g2, hw, dinv, b2, out_ch, rows_blk):
    n, d_pad = hw.shape
    grid = n // rows_blk

    def body(g_ref, hw_ref, dinv_ref, b2_ref, out_ref):
        gsum = jnp.sum(g_ref[...], axis=0)
        zf = gsum + dinv_ref[...] * hw_ref[...]
        z = zf[:, :out_ch] + b2_ref[0, :]
        m = jnp.max(z, axis=1, keepdims=True)
        e = z - m
        lse = jnp.log(jnp.sum(jnp.exp(e), axis=1, keepdims=True))
        out_ref[...] = e - lse

    return pl.pallas_call(
        body,
        grid=(grid,),
        in_specs=[
            pl.BlockSpec((NC, rows_blk, d_pad), lambda i: (0, i, 0)),
            pl.BlockSpec((rows_blk, d_pad), lambda i: (i, 0)),
            pl.BlockSpec((rows_blk, 1), lambda i: (i, 0)),
            pl.BlockSpec((1, out_ch), lambda i: (0, 0)),
        ],
        out_specs=pl.BlockSpec((rows_blk, out_ch), lambda i: (i, 0)),
        out_shape=jax.ShapeDtypeStruct((n, out_ch), jnp.float32),
    )(g2, hw, dinv, b2)


@jax.jit
def kernel(x, edge_index, edge_weight, W1, b1, W2, b2):
    n, _ = x.shape
    hid = W1.shape[0]
    out_ch = W2.shape[0]
    e = edge_index.shape[1]
    d_pad = out_ch

    src = edge_index[0].astype(jnp.int32)
    dst = edge_index[1].astype(jnp.int32)
    ew = edge_weight.astype(jnp.float32)

    ep = ((e + NC * NS * CB - 1) // (NC * NS * CB)) * (NC * NS * CB)
    pad = ep - e
    if pad:
        src = jnp.concatenate([src, jnp.zeros((pad,), jnp.int32)])
        dst = jnp.concatenate([dst, jnp.zeros((pad,), jnp.int32)])
        ew = jnp.concatenate([ew, jnp.zeros((pad,), jnp.float32)])
    src2 = src.reshape(ep // CB, CB)
    dst2 = dst.reshape(ep // CB, CB)
    ew2 = ew.reshape(ep // CB, CB)

    rows_blk = 1000
    n_pad = ((n + NS * CB - 1) // (NS * CB)) * (NS * CB)

    deg32 = _make_deg_kernel(n_pad, ep)(dst2, ew2)
    dis_r, dinv_r = _tc_degnorm(deg32.reshape(NC * NS, n_pad))
    dis = dis_r.reshape(n_pad)[:n]
    dinv = dinv_r.reshape(n_pad)[:n].reshape(n, 1)

    xa, xb = _tc1(x, W1.T, rows_blk)
    aa, ab = _make_agg_feat_split(n, n_pad, ep, hid // 2)(
        xa, xb, src2, dst2, ew2, dis)
    hw = _tc2(aa, ab, xa, xb, dinv, b1.reshape(1, hid), W2.T, d_pad, rows_blk)
    g2 = _make_agg_edge_split(n, n_pad, ep, d_pad)(
        hw, src2, dst2, ew2, dis)
    return _tc3(g2, hw, dinv, b2.reshape(1, out_ch), out_ch, rows_blk)
